# Initial kernel scaffold; baseline (speedup 1.0000x reference)
import jax
import jax.numpy as jnp
from jax.experimental import pallas as pl


def kernel(x, seg, tok_table, pos_table, seg_table, gamma, beta):
    raise NotImplementedError("write your pallas kernel here")


# speedup vs baseline: 3.4817x; 3.4817x over previous
import functools

import jax
import jax.numpy as jnp
from jax import lax
from jax.experimental import pallas as pl
from jax.experimental.pallas import tpu as pltpu
from jax.experimental.pallas import tpu_sc as plsc

_EPS = 1e-5
_LANES = 16


def _sc_geometry():
    try:
        info = plsc.get_sparse_core_info()
        return info.num_cores, info.num_subcores
    except Exception:
        return 2, 16


def _build_table(tok, pos, seg, gamma, beta, seq_len):
    V, D = tok.shape
    NS = seg.shape[0]

    def body(tok_ref, pos_ref, seg_ref, gam_ref, bet_ref, out_ref):
        tokv = tok_ref[...]
        posv = pos_ref[:seq_len, :]
        segv = seg_ref[...]
        e = (tokv[:, None, None, :] + posv[None, None, :, :]) + segv[None, :, None, :]
        mean = jnp.mean(e, axis=-1, keepdims=True)
        c = e - mean
        var = jnp.mean(c * c, axis=-1, keepdims=True)
        y = c * lax.rsqrt(var + _EPS)
        y = y * gam_ref[...] + bet_ref[...]
        out_ref[...] = y.reshape(V * NS, seq_len, D)

    out = pl.pallas_call(
        body,
        out_shape=jax.ShapeDtypeStruct((V * NS, seq_len, D), jnp.float32),
    )(tok, pos, seg, gamma.reshape(1, -1), beta.reshape(1, -1))
    return out.reshape(V * NS * seq_len, D)


def _sc_gather(lnt, xf, sf, seq_len, nseg):
    T, D = lnt.shape
    BT = xf.shape[0]
    NC, NSUB = _sc_geometry()
    NW = NC * NSUB
    assert BT % (NW * _LANES) == 0
    per_w = BT // NW
    CH = 64
    assert per_w % (2 * CH) == 0
    nch = per_w // CH

    mesh = plsc.VectorSubcoreMesh(
        core_axis_name="c", subcore_axis_name="s",
        num_cores=NC, num_subcores=NSUB)

    @functools.partial(
        pl.kernel,
        out_type=jax.ShapeDtypeStruct((BT, D), jnp.float32),
        mesh=mesh,
        scratch_types=[
            pltpu.VMEM((per_w,), jnp.int32),
            pltpu.VMEM((per_w,), jnp.int32),
            pltpu.VMEM((per_w,), jnp.int32),
            pltpu.VMEM((CH, D), jnp.float32),
            pltpu.VMEM((CH, D), jnp.float32),
            pltpu.SemaphoreType.DMA,
            pltpu.SemaphoreType.DMA,
            pltpu.SemaphoreType.DMA,
            pltpu.SemaphoreType.DMA,
        ],
    )
    def k(x_hbm, s_hbm, lnt_hbm, out_hbm, xv, sv, iv, buf0, buf1,
          gs0, gs1, ts0, ts1):
        wid = lax.axis_index("s") * NC + lax.axis_index("c")
        base = wid * per_w
        pltpu.sync_copy(x_hbm.at[pl.ds(base, per_w)], xv)
        pltpu.sync_copy(s_hbm.at[pl.ds(base, per_w)], sv)

        lane = lax.iota(jnp.int32, _LANES)

        @pl.loop(0, per_w // _LANES)
        def _(i):
            off = i * _LANES
            xi = xv[pl.ds(off, _LANES)]
            si = sv[pl.ds(off, _LANES)]
            j = lax.rem(base + off + lane, seq_len)
            iv[pl.ds(off, _LANES)] = (xi * nseg + si) * seq_len + j

        def g_start(c, buf, sem):
            pltpu.async_copy(lnt_hbm.at[iv.at[pl.ds(c * CH, CH)]], buf, sem)

        def g_wait(buf, sem):
            pltpu.make_async_copy(lnt_hbm.at[pl.ds(0, CH)], buf, sem).wait()

        def s_start(c, buf, sem):
            return pltpu.async_copy(buf, out_hbm.at[pl.ds(base + c * CH, CH)],
                                    sem)

        g_start(0, buf0, gs0)
        g_start(1, buf1, gs1)

        @pl.loop(0, nch // 2)
        def _(t):
            c0 = 2 * t
            c1 = c0 + 1
            g_wait(buf0, gs0)
            st0 = s_start(c0, buf0, ts0)
            g_wait(buf1, gs1)
            st1 = s_start(c1, buf1, ts1)
            st0.wait()

            @pl.when(c0 + 2 < nch)
            def _():
                g_start(c0 + 2, buf0, gs0)

            st1.wait()

            @pl.when(c1 + 2 < nch)
            def _():
                g_start(c1 + 2, buf1, gs1)

    return k(xf, sf, lnt)


def kernel(x, seg, tok_table, pos_table, seg_table, gamma, beta):
    B, S = x.shape
    NS = seg_table.shape[0]
    D = tok_table.shape[1]
    lnt = _build_table(tok_table, pos_table, seg_table, gamma, beta, S)
    xf = x.reshape(-1).astype(jnp.int32)
    sf = seg.reshape(-1).astype(jnp.int32)
    out = _sc_gather(lnt, xf, sf, S, NS)
    return out.reshape(B, S, D)


# speedup vs baseline: 3.4874x; 1.0016x over previous
import functools

import jax
import jax.numpy as jnp
from jax import lax
from jax.experimental import pallas as pl
from jax.experimental.pallas import tpu as pltpu
from jax.experimental.pallas import tpu_sc as plsc

_EPS = 1e-5
_LANES = 16


def _sc_geometry():
    try:
        info = plsc.get_sparse_core_info()
        return info.num_cores, info.num_subcores
    except Exception:
        return 2, 16


def _build_table(tok, pos, seg, gamma, beta, seq_len):
    V, D = tok.shape
    NS = seg.shape[0]

    def body(tok_ref, pos_ref, seg_ref, gam_ref, bet_ref, out_ref):
        tokv = tok_ref[...]
        posv = pos_ref[:seq_len, :]
        segv = seg_ref[...]
        e = (tokv[:, None, None, :] + posv[None, None, :, :]) + segv[None, :, None, :]
        mean = jnp.mean(e, axis=-1, keepdims=True)
        c = e - mean
        var = jnp.mean(c * c, axis=-1, keepdims=True)
        y = c * lax.rsqrt(var + _EPS)
        y = y * gam_ref[...] + bet_ref[...]
        out_ref[...] = y.reshape(V * NS, seq_len, D)

    out = pl.pallas_call(
        body,
        out_shape=jax.ShapeDtypeStruct((V * NS, seq_len, D), jnp.float32),
    )(tok, pos, seg, gamma.reshape(1, -1), beta.reshape(1, -1))
    return out.reshape(V * NS * seq_len, D)


def _sc_gather(lnt, xf, sf, seq_len, nseg):
    T, D = lnt.shape
    BT = xf.shape[0]
    NC, NSUB = _sc_geometry()
    NW = NC * NSUB
    assert BT % (NW * _LANES) == 0
    per_w = BT // NW
    CH = 32
    NBUF = 4
    assert per_w % (NBUF * CH) == 0
    nch = per_w // CH

    mesh = plsc.VectorSubcoreMesh(
        core_axis_name="c", subcore_axis_name="s",
        num_cores=NC, num_subcores=NSUB)

    @functools.partial(
        pl.kernel,
        out_type=jax.ShapeDtypeStruct((BT, D), jnp.float32),
        mesh=mesh,
        scratch_types=[
            pltpu.VMEM((per_w,), jnp.int32),
            pltpu.VMEM((per_w,), jnp.int32),
            pltpu.VMEM((per_w,), jnp.int32),
            [pltpu.VMEM((CH, D), jnp.float32) for _ in range(NBUF)],
            [pltpu.SemaphoreType.DMA for _ in range(NBUF)],
            [pltpu.SemaphoreType.DMA for _ in range(NBUF)],
        ],
    )
    def k(x_hbm, s_hbm, lnt_hbm, out_hbm, xv, sv, iv, bufs, gsems, tsems):
        sid = lax.axis_index("s")
        wid = sid * NC + lax.axis_index("c")
        base = wid * per_w

        pltpu.sync_copy(x_hbm.at[pl.ds(base, per_w)], xv)
        pltpu.sync_copy(s_hbm.at[pl.ds(base, per_w)], sv)

        lane = lax.iota(jnp.int32, _LANES)

        @pl.loop(0, per_w // _LANES)
        def _(i):
            off = i * _LANES
            xi = xv[pl.ds(off, _LANES)]
            si = sv[pl.ds(off, _LANES)]
            j = lax.rem(base + off + lane, seq_len)
            iv[pl.ds(off, _LANES)] = (xi * nseg + si) * seq_len + j

        def g_start(c, buf, sem):
            pltpu.async_copy(lnt_hbm.at[iv.at[pl.ds(c * CH, CH)]], buf, sem)

        def g_wait(buf, sem):
            pltpu.make_async_copy(lnt_hbm.at[pl.ds(0, CH)], buf, sem).wait()

        def s_start(c, buf, sem):
            return pltpu.async_copy(buf, out_hbm.at[pl.ds(base + c * CH, CH)],
                                    sem)

        for b in range(NBUF):
            g_start(b, bufs[b], gsems[b])

        @pl.loop(0, nch // NBUF)
        def _(t):
            c0 = t * NBUF
            sts = []
            for b in range(NBUF):
                g_wait(bufs[b], gsems[b])
                sts.append(s_start(c0 + b, bufs[b], tsems[b]))
            for b in range(NBUF):
                sts[b].wait()

                @pl.when(c0 + b + NBUF < nch)
                def _(b=b):
                    g_start(c0 + b + NBUF, bufs[b], gsems[b])

    return k(xf, sf, lnt)


def kernel(x, seg, tok_table, pos_table, seg_table, gamma, beta):
    B, S = x.shape
    NS = seg_table.shape[0]
    D = tok_table.shape[1]
    lnt = _build_table(tok_table, pos_table, seg_table, gamma, beta, S)
    xf = x.reshape(-1).astype(jnp.int32)
    sf = seg.reshape(-1).astype(jnp.int32)
    out = _sc_gather(lnt, xf, sf, S, NS)
    return out.reshape(B, S, D)
